# Initial kernel scaffold; baseline (speedup 1.0000x reference)
import jax
import jax.numpy as jnp
from jax.experimental import pallas as pl


def kernel(nf1_data, class_table):
    raise NotImplementedError("write your pallas kernel here")


# speedup vs baseline: 4.2100x; 4.2100x over previous
import functools

import jax
import jax.numpy as jnp
from jax import lax
from jax.experimental import pallas as pl
from jax.experimental.pallas import tpu as pltpu
from jax.experimental.pallas import tpu_sc as plsc

EMB_DIM = 64
K = 8
B = 4096
HALF = EMB_DIM // 2
D = K * EMB_DIM
R = 2 * B

NC = 2
NS = 16
NW = NC * NS
ROWS_PER_W = R // NW
CHUNK = 128

BLK = 128
NBLK = B // BLK


@functools.cache
def _make_sc_gather():
    mesh = plsc.VectorSubcoreMesh(
        core_axis_name="c", subcore_axis_name="s", num_cores=NC, num_subcores=NS
    )

    @functools.partial(
        pl.kernel,
        mesh=mesh,
        out_type=jax.ShapeDtypeStruct((R, D), jnp.float32),
        scratch_types=[
            pltpu.VMEM((CHUNK,), jnp.int32),
            pltpu.VMEM((CHUNK, D), jnp.float32),
            pltpu.SemaphoreType.DMA,
        ],
    )
    def gather_k(idx_hbm, table_hbm, out_hbm, idx_v, rows_v, sem):
        wid = lax.axis_index("s") * NC + lax.axis_index("c")
        base = wid * ROWS_PER_W
        for ch in range(ROWS_PER_W // CHUNK):
            off = base + ch * CHUNK
            pltpu.sync_copy(idx_hbm.at[pl.ds(off, CHUNK)], idx_v)
            pltpu.async_copy(table_hbm.at[idx_v], rows_v, sem).wait()
            pltpu.sync_copy(rows_v, out_hbm.at[pl.ds(off, CHUNK), :])

    return gather_k


def _prod_major(x):
    n = x.shape[0]
    while n > 1:
        n //= 2
        x = x[:n] * x[n:]
    return x


def _tc_body(c_ref, d_ref, o_ref):
    i = pl.program_id(0)
    c = c_ref[...]
    d = d_ref[...]
    cc, co = c[:HALF], jnp.abs(c[HALF:])
    dc, do = d[:HALF], jnp.abs(d[HALF:])
    c_lo, c_hi = cc - co, cc + co
    d_lo, d_hi = dc - do, dc + do

    lo = jnp.maximum(c_lo[:, :, None, :], d_lo[:, None, :, :])
    hi = jnp.minimum(c_hi[:, :, None, :], d_hi[:, None, :, :])
    side = jnp.maximum(hi - lo, 0.0)
    inter_area = jnp.sum(_prod_major(side)[0], axis=(0, 1))

    c_area = jnp.sum(_prod_major(2.0 * co)[0], axis=0)

    loses = jnp.where(
        c_area == 0.0,
        0.0,
        jnp.where(
            jnp.isinf(c_area),
            1.0 - inter_area * 0.5,
            1.0 - inter_area / c_area,
        ),
    )
    r = jnp.maximum(loses, 0.0)
    partial = jnp.sum(r * r)

    @pl.when(i == 0)
    def _init():
        o_ref[0, 0] = 0.0

    o_ref[0, 0] += partial

    @pl.when(i == NBLK - 1)
    def _fin():
        o_ref[0, 0] = jnp.sqrt(o_ref[0, 0])


_tc_call = pl.pallas_call(
    _tc_body,
    grid=(NBLK,),
    in_specs=[
        pl.BlockSpec((EMB_DIM, K, BLK), lambda i: (0, 0, i)),
        pl.BlockSpec((EMB_DIM, K, BLK), lambda i: (0, 0, i + NBLK)),
    ],
    out_specs=pl.BlockSpec((1, 1), lambda i: (0, 0), memory_space=pltpu.SMEM),
    out_shape=jax.ShapeDtypeStruct((1, 1), jnp.float32),
    compiler_params=pltpu.CompilerParams(
        dimension_semantics=("arbitrary",),
    ),
)


def kernel(nf1_data, class_table):
    flat_idx = nf1_data.astype(jnp.int32).T.reshape(R)
    gathered = _make_sc_gather()(flat_idx, class_table)
    t2 = gathered.reshape(R, K, EMB_DIM).transpose(2, 1, 0)
    res = _tc_call(t2, t2)
    return res[0, 0]


# speedup vs baseline: 6.6046x; 1.5688x over previous
import functools

import jax
import jax.numpy as jnp
from jax import lax
from jax.experimental import pallas as pl
from jax.experimental.pallas import tpu as pltpu
from jax.experimental.pallas import tpu_sc as plsc

EMB_DIM = 64
K = 8
B = 4096
HALF = EMB_DIM // 2
D = K * EMB_DIM
R = 2 * B

NC = 2
NS = 16
NW = NC * NS
ROWS_PER_W = R // NW
CHUNK = 128

BLK = 128
NBLK = B // BLK


@functools.cache
def _make_sc_gather():
    mesh = plsc.VectorSubcoreMesh(
        core_axis_name="c", subcore_axis_name="s", num_cores=NC, num_subcores=NS
    )

    @functools.partial(
        pl.kernel,
        mesh=mesh,
        out_type=jax.ShapeDtypeStruct((R, D), jnp.float32),
        scratch_types=[
            pltpu.VMEM((CHUNK,), jnp.int32),
            pltpu.VMEM((CHUNK, D), jnp.float32),
            pltpu.SemaphoreType.DMA,
        ],
    )
    def gather_k(idx_hbm, table_hbm, out_hbm, idx_v, rows_v, sem):
        wid = lax.axis_index("s") * NC + lax.axis_index("c")
        base = wid * ROWS_PER_W
        for ch in range(ROWS_PER_W // CHUNK):
            off = base + ch * CHUNK
            pltpu.sync_copy(idx_hbm.at[pl.ds(off, CHUNK)], idx_v)
            pltpu.async_copy(table_hbm.at[idx_v], rows_v, sem).wait()
            pltpu.sync_copy(rows_v, out_hbm.at[pl.ds(off, CHUNK), :])

    return gather_k


def _prod_sub(x):
    n = x.shape[1]
    while n > 8:
        n //= 2
        x = x[:, :n] * x[:, n:]
    x = x * pltpu.roll(x, 4, 1)
    x = x * pltpu.roll(x, 2, 1)
    x = x * pltpu.roll(x, 1, 1)
    return x


def _tc_body(c_ref, d_ref, o_ref):
    i = pl.program_id(0)
    c = c_ref[...].T.reshape(K, EMB_DIM, BLK)
    d = d_ref[...].T.reshape(K, EMB_DIM, BLK)
    cc, co = c[:, :HALF], jnp.abs(c[:, HALF:])
    dc, do = d[:, :HALF], jnp.abs(d[:, HALF:])
    c_lo, c_hi = cc - co, cc + co
    d_lo, d_hi = dc - do, dc + do

    inter8 = jnp.zeros((K, BLK), jnp.float32)
    for bi in range(K):
        lo = jnp.maximum(c_lo[bi][None], d_lo)
        hi = jnp.minimum(c_hi[bi][None], d_hi)
        side = jnp.maximum(hi - lo, 0.0)
        p = _prod_sub(side)
        inter8 = inter8 + jnp.sum(p, axis=0)
    inter_area = inter8

    ca = _prod_sub(2.0 * co)
    c_area = jnp.sum(ca, axis=0)

    loses = jnp.where(
        c_area == 0.0,
        0.0,
        jnp.where(
            jnp.isinf(c_area),
            1.0 - inter_area * 0.5,
            1.0 - inter_area / c_area,
        ),
    )
    r = jnp.maximum(loses, 0.0)
    partial = jnp.sum(r * r) * 0.125

    @pl.when(i == 0)
    def _init():
        o_ref[0, 0] = 0.0

    o_ref[0, 0] += partial

    @pl.when(i == NBLK - 1)
    def _fin():
        o_ref[0, 0] = jnp.sqrt(o_ref[0, 0])


_tc_call = pl.pallas_call(
    _tc_body,
    grid=(NBLK,),
    in_specs=[
        pl.BlockSpec((BLK, D), lambda i: (i, 0)),
        pl.BlockSpec((BLK, D), lambda i: (i + NBLK, 0)),
    ],
    out_specs=pl.BlockSpec((1, 1), lambda i: (0, 0), memory_space=pltpu.SMEM),
    out_shape=jax.ShapeDtypeStruct((1, 1), jnp.float32),
    compiler_params=pltpu.CompilerParams(
        dimension_semantics=("arbitrary",),
    ),
)


def kernel(nf1_data, class_table):
    flat_idx = nf1_data.astype(jnp.int32).T.reshape(R)
    gathered = _make_sc_gather()(flat_idx, class_table)
    res = _tc_call(gathered, gathered)
    return res[0, 0]


# speedup vs baseline: 7.5567x; 1.1442x over previous
import functools

import jax
import jax.numpy as jnp
from jax import lax
from jax.experimental import pallas as pl
from jax.experimental.pallas import tpu as pltpu
from jax.experimental.pallas import tpu_sc as plsc

EMB_DIM = 64
K = 8
B = 4096
HALF = EMB_DIM // 2
D = K * EMB_DIM
R = 2 * B

NC = 2
NS = 16
NW = NC * NS
EX_PER_W = B // NW
CHUNK = 64

BLK = 256
NBLK = B // BLK


@functools.cache
def _make_sc_gather():
    mesh = plsc.VectorSubcoreMesh(
        core_axis_name="c", subcore_axis_name="s", num_cores=NC, num_subcores=NS
    )

    @functools.partial(
        pl.kernel,
        mesh=mesh,
        out_type=jax.ShapeDtypeStruct((R, D), jnp.float32),
        scratch_types=[
            pltpu.VMEM((2 * EX_PER_W,), jnp.int32),
            pltpu.VMEM((CHUNK, D), jnp.float32),
            pltpu.VMEM((CHUNK, D), jnp.float32),
            pltpu.SemaphoreType.DMA,
            pltpu.SemaphoreType.DMA,
            pltpu.SemaphoreType.DMA,
            pltpu.SemaphoreType.DMA,
        ],
    )
    def gather_k(idx_hbm, table_hbm, out_hbm, idx_v, buf0, buf1, g0, g1, w0, w1):
        wid = lax.axis_index("s") * NC + lax.axis_index("c")
        r0 = wid * 2 * EX_PER_W
        pltpu.sync_copy(idx_hbm.at[pl.ds(r0, 2 * EX_PER_W)], idx_v)
        bufs, gsem, wsem = (buf0, buf1), (g0, g1), (w0, w1)
        writes = [None, None]
        for k in range(2 * EX_PER_W // CHUNK):
            bsel = k % 2
            if writes[bsel] is not None:
                writes[bsel].wait()
            pltpu.async_copy(
                table_hbm.at[idx_v.at[pl.ds(k * CHUNK, CHUNK)]],
                bufs[bsel],
                gsem[bsel],
            ).wait()
            writes[bsel] = pltpu.async_copy(
                bufs[bsel], out_hbm.at[pl.ds(r0 + k * CHUNK, CHUNK)], wsem[bsel]
            )
        writes[0].wait()
        writes[1].wait()

    return gather_k


def _prod_sub(x):
    n = x.shape[1]
    while n > 8:
        n //= 2
        x = x[:, :n] * x[:, n:]
    x = x * pltpu.roll(x, 4, 1)
    x = x * pltpu.roll(x, 2, 1)
    x = x * pltpu.roll(x, 1, 1)
    return x


def _tc_body(c_ref, d_ref, o_ref):
    i = pl.program_id(0)
    c = c_ref[...].T.reshape(K, EMB_DIM, BLK)
    d = d_ref[...].T.reshape(K, EMB_DIM, BLK)
    cc, co = c[:, :HALF], jnp.abs(c[:, HALF:])
    dc, do = d[:, :HALF], jnp.abs(d[:, HALF:])
    c_lo, c_hi = cc - co, cc + co
    d_lo, d_hi = dc - do, dc + do

    inter8 = jnp.zeros((K, BLK), jnp.float32)
    for bi in range(K):
        lo = jnp.maximum(c_lo[bi][None], d_lo)
        hi = jnp.minimum(c_hi[bi][None], d_hi)
        side = jnp.maximum(hi - lo, 0.0)
        p = _prod_sub(side)
        inter8 = inter8 + jnp.sum(p, axis=0)
    inter_area = inter8

    ca = _prod_sub(2.0 * co)
    c_area = jnp.sum(ca, axis=0)

    loses = jnp.where(
        c_area == 0.0,
        0.0,
        jnp.where(
            jnp.isinf(c_area),
            1.0 - inter_area * 0.5,
            1.0 - inter_area / c_area,
        ),
    )
    r = jnp.maximum(loses, 0.0)
    partial = jnp.sum(r * r) * 0.125

    @pl.when(i == 0)
    def _init():
        o_ref[0, 0] = 0.0

    o_ref[0, 0] += partial

    @pl.when(i == NBLK - 1)
    def _fin():
        o_ref[0, 0] = jnp.sqrt(o_ref[0, 0])


_tc_call = pl.pallas_call(
    _tc_body,
    grid=(NBLK,),
    in_specs=[
        pl.BlockSpec((BLK, D), lambda i: (i, 0)),
        pl.BlockSpec((BLK, D), lambda i: (i + NBLK, 0)),
    ],
    out_specs=pl.BlockSpec((1, 1), lambda i: (0, 0), memory_space=pltpu.SMEM),
    out_shape=jax.ShapeDtypeStruct((1, 1), jnp.float32),
    compiler_params=pltpu.CompilerParams(
        dimension_semantics=("arbitrary",),
    ),
)


def kernel(nf1_data, class_table):
    flat_idx = nf1_data.astype(jnp.int32).T.reshape(R)
    gathered = _make_sc_gather()(flat_idx, class_table)
    res = _tc_call(gathered, gathered)
    return res[0, 0]


# speedup vs baseline: 8.1536x; 1.0790x over previous
import functools

import jax
import jax.numpy as jnp
from jax import lax
from jax.experimental import pallas as pl
from jax.experimental.pallas import tpu as pltpu
from jax.experimental.pallas import tpu_sc as plsc

EMB_DIM = 64
K = 8
B = 4096
HALF = EMB_DIM // 2
D = K * EMB_DIM
R = 2 * B

NC = 2
NS = 16
NW = NC * NS
EX_PER_W = B // NW
CHUNK = 64

BLK = 512
NBLK = B // BLK


@functools.cache
def _make_sc_gather():
    mesh = plsc.VectorSubcoreMesh(
        core_axis_name="c", subcore_axis_name="s", num_cores=NC, num_subcores=NS
    )

    @functools.partial(
        pl.kernel,
        mesh=mesh,
        out_type=jax.ShapeDtypeStruct((R, D), jnp.float32),
        scratch_types=[
            pltpu.VMEM((2 * EX_PER_W,), jnp.int32),
            pltpu.VMEM((CHUNK, D), jnp.float32),
            pltpu.VMEM((CHUNK, D), jnp.float32),
            pltpu.SemaphoreType.DMA,
            pltpu.SemaphoreType.DMA,
            pltpu.SemaphoreType.DMA,
            pltpu.SemaphoreType.DMA,
        ],
    )
    def gather_k(idx_hbm, table_hbm, out_hbm, idx_v, buf0, buf1, g0, g1, w0, w1):
        wid = lax.axis_index("s") * NC + lax.axis_index("c")
        r0 = wid * 2 * EX_PER_W
        pltpu.sync_copy(idx_hbm.at[pl.ds(r0, 2 * EX_PER_W)], idx_v)
        bufs, gsem, wsem = (buf0, buf1), (g0, g1), (w0, w1)
        writes = [None, None]
        for k in range(2 * EX_PER_W // CHUNK):
            bsel = k % 2
            if writes[bsel] is not None:
                writes[bsel].wait()
            pltpu.async_copy(
                table_hbm.at[idx_v.at[pl.ds(k * CHUNK, CHUNK)]],
                bufs[bsel],
                gsem[bsel],
            ).wait()
            writes[bsel] = pltpu.async_copy(
                bufs[bsel], out_hbm.at[pl.ds(r0 + k * CHUNK, CHUNK)], wsem[bsel]
            )
        writes[0].wait()
        writes[1].wait()

    return gather_k


def _prod_sub(x):
    n = x.shape[1]
    while n > 8:
        n //= 2
        x = x[:, :n] * x[:, n:]
    x = x * pltpu.roll(x, 4, 1)
    x = x * pltpu.roll(x, 2, 1)
    x = x * pltpu.roll(x, 1, 1)
    return x


def _tc_body(c_ref, d_ref, o_ref):
    i = pl.program_id(0)
    c = c_ref[...].T.reshape(K, EMB_DIM, BLK)
    d = d_ref[...].T.reshape(K, EMB_DIM, BLK)
    cc, co = c[:, :HALF], jnp.abs(c[:, HALF:])
    dc, do = d[:, :HALF], jnp.abs(d[:, HALF:])
    c_lo, c_hi = cc - co, cc + co
    d_lo, d_hi = dc - do, dc + do

    inter8 = jnp.zeros((K, BLK), jnp.float32)
    for bi in range(K):
        lo = jnp.maximum(c_lo[bi][None], d_lo)
        hi = jnp.minimum(c_hi[bi][None], d_hi)
        side = jnp.maximum(hi - lo, 0.0)
        p = _prod_sub(side)
        inter8 = inter8 + jnp.sum(p, axis=0)
    inter_area = inter8

    ca = _prod_sub(2.0 * co)
    c_area = jnp.sum(ca, axis=0)

    loses = jnp.where(
        c_area == 0.0,
        0.0,
        jnp.where(
            jnp.isinf(c_area),
            1.0 - inter_area * 0.5,
            1.0 - inter_area / c_area,
        ),
    )
    r = jnp.maximum(loses, 0.0)
    partial = jnp.sum(r * r) * 0.125

    @pl.when(i == 0)
    def _init():
        o_ref[0, 0] = 0.0

    o_ref[0, 0] += partial

    @pl.when(i == NBLK - 1)
    def _fin():
        o_ref[0, 0] = jnp.sqrt(o_ref[0, 0])


_tc_call = pl.pallas_call(
    _tc_body,
    grid=(NBLK,),
    in_specs=[
        pl.BlockSpec((BLK, D), lambda i: (i, 0)),
        pl.BlockSpec((BLK, D), lambda i: (i + NBLK, 0)),
    ],
    out_specs=pl.BlockSpec((1, 1), lambda i: (0, 0), memory_space=pltpu.SMEM),
    out_shape=jax.ShapeDtypeStruct((1, 1), jnp.float32),
    compiler_params=pltpu.CompilerParams(
        dimension_semantics=("arbitrary",),
    ),
)


def kernel(nf1_data, class_table):
    flat_idx = nf1_data.astype(jnp.int32).T.reshape(R)
    gathered = _make_sc_gather()(flat_idx, class_table)
    res = _tc_call(gathered, gathered)
    return res[0, 0]
